# baseline (speedup 1.0000x reference)
import functools

import jax
import jax.numpy as jnp
from jax import lax
from jax.experimental import pallas as pl
from jax.experimental.pallas import tpu as pltpu
from jax.experimental.pallas import tpu_sc as plsc

N = 10000
E = 320000
D = 128
L = 2
T = 2

NC = 2
NS = 16
NW = NC * NS
NP = 10240
RPT = NP // NS
EPW = E // NW
CH = 80
NCH = EPW // CH
RZ = 80
BLK = 1000
GRID = N // BLK


def _make_agg(num_feats, with_count):
    mesh = plsc.VectorSubcoreMesh(core_axis_name="c", subcore_axis_name="s")
    out_type = [jax.ShapeDtypeStruct((NC, NP, D), jnp.float32)
                for _ in range(num_feats + (1 if with_count else 0))]
    scratch = [
        pltpu.VMEM((EPW,), jnp.int32),
        pltpu.VMEM((CH,), jnp.int32),
        pltpu.VMEM((CH,), jnp.int32),
        pltpu.VMEM((CH,), jnp.int32),
        pltpu.VMEM((CH, D), jnp.float32),
        pltpu.VMEM((CH, D), jnp.float32),
        pltpu.VMEM((CH, D), jnp.float32),
        pltpu.VMEM_SHARED((NP, D), jnp.float32),
        pltpu.SemaphoreType.DMA,
        pltpu.SemaphoreType.DMA,
        pltpu.SemaphoreType.DMA,
        pltpu.SemaphoreType.DMA,
        pltpu.SemaphoreType.DMA,
        pltpu.SemaphoreType.DMA,
        pltpu.SemaphoreType.DMA,
        pltpu.SemaphoreType.DMA,
        pltpu.SemaphoreType.DMA,
    ]

    @functools.partial(pl.kernel, mesh=mesh, out_type=tuple(out_type),
                       scratch_types=tuple(scratch))
    def agg(*refs):
        feats = refs[:num_feats]
        src_h = refs[num_feats]
        dst_h = refs[num_feats + 1]
        n_out = num_feats + (1 if with_count else 0)
        outs = refs[num_feats + 2: num_feats + 2 + n_out]
        (srcb, db0, db1, db2, rows0, rows1, rows2, accum,
         sg0, sg1, sg2, ss0, ss1, ss2, sd0, sd1, sd2) = \
            refs[num_feats + 2 + n_out:]
        rowsb = (rows0, rows1, rows2)
        dstb = (db0, db1, db2)
        sgs = (sg0, sg1, sg2)
        sss = (ss0, ss1, ss2)
        sds = (sd0, sd1, sd2)

        c = lax.axis_index("c")
        s = lax.axis_index("s")
        wid = s * NC + c
        ebase = wid * EPW
        base = s * RPT

        pltpu.sync_copy(src_h.at[pl.ds(ebase, EPW)], srcb)

        def fill_rows(rb, val):
            def zrow(i, _):
                for l in range(D // 16):
                    rb[i, pl.ds(l * 16, 16)] = jnp.full((16,), val,
                                                        jnp.float32)
                return 0
            lax.fori_loop(0, RZ, zrow, 0)

        def zero_stripe():
            fill_rows(rows0, 0.0)
            for k in range(RPT // RZ):
                pltpu.sync_copy(rows0, accum.at[pl.ds(base + k * RZ, RZ)])

        def g_start(f, j, b):
            pltpu.async_copy(feats[f].at[srcb.at[pl.ds(j * CH, CH)]],
                             rowsb[b], sgs[b])

        def g_wait(f, j, b):
            pltpu.make_async_copy(
                feats[f].at[srcb.at[pl.ds(j * CH, CH)]],
                rowsb[b], sgs[b]).wait()

        def sc_start(j, b):
            pltpu.async_copy(rowsb[b], accum.at[dstb[b]], sss[b], add=True)

        def sc_wait(j, b):
            pltpu.make_async_copy(rowsb[b], accum.at[dstb[b]],
                                  sss[b]).wait()

        def d_start(j, b):
            pltpu.async_copy(dst_h.at[pl.ds(ebase + j * CH, CH)],
                             dstb[b], sds[b])

        def d_wait(j, b):
            pltpu.make_async_copy(dst_h.at[pl.ds(ebase + j * CH, CH)],
                                  dstb[b], sds[b]).wait()

        zero_stripe()
        plsc.subcore_barrier()

        for f in range(n_out):
            ones = f >= num_feats
            if ones:
                for rb in rowsb:
                    fill_rows(rb, 1.0)

            def step(f, j, b, wait_prev, start_next):
                if not ones:
                    g_wait(f, j, b)
                d_wait(j, b)
                sc_start(j, b)
                if wait_prev:
                    sc_wait(j - 2, (j + 1) % 3)
                if start_next:
                    if not ones:
                        g_start(f, j + 1, (j + 1) % 3)
                    d_start(j + 1, (j + 1) % 3)

            if not ones:
                g_start(f, 0, 0)
            d_start(0, 0)
            step(f, 0, 0, False, True)
            step(f, 1, 1, False, True)

            def tri(i, _):
                for q in range(3):
                    j = 3 * i + 2 + q
                    b = (2 + q) % 3
                    if not ones:
                        g_wait(f, j, b)
                    d_wait(j, b)
                    sc_start(j, b)
                    sc_wait(j - 2, q)
                    if not ones:
                        g_start(f, j + 1, q)
                    d_start(j + 1, q)
                return 0
            lax.fori_loop(0, (NCH - 5) // 3, tri, 0)
            step(f, NCH - 3, (NCH - 3) % 3, True, True)
            step(f, NCH - 2, (NCH - 2) % 3, True, True)
            step(f, NCH - 1, (NCH - 1) % 3, True, False)
            sc_wait(NCH - 2, (NCH - 2) % 3)
            sc_wait(NCH - 1, (NCH - 1) % 3)
            plsc.subcore_barrier()

            for k in range(RPT // RZ):
                r0 = base + k * RZ
                pltpu.sync_copy(accum.at[pl.ds(r0, RZ)], rows0)
                pltpu.sync_copy(rows0, outs[f].at[c, pl.ds(r0, RZ)])
            if f + 1 < n_out:
                zero_stripe()
                plsc.subcore_barrier()

    return agg


_agg = functools.lru_cache(None)(_make_agg)


def _gate_body(final, *refs):
    if final:
        (inp, h, cprev, axp, ahp, cntp, wxs, wxn, whs, whn, wcj, ball,
         wout, bout, nh, ncell, out) = refs
    else:
        (inp, h, cprev, axp, ahp, cntp, wxs, wxn, whs, whn, wcj, ball,
         nh, ncell) = refs
    cnt = cntp[0, :, 0:1] + cntp[1, :, 0:1]
    inv = 1.0 / jnp.maximum(cnt, 1.0)
    ax = (axp[0] + axp[1]) * inv
    ah = (ahp[0] + ahp[1]) * inv
    g = (jnp.dot(inp[...], wxs[...], preferred_element_type=jnp.float32)
         + jnp.dot(ax, wxn[...], preferred_element_type=jnp.float32)
         + jnp.dot(h[...], whs[...], preferred_element_type=jnp.float32)
         + jnp.dot(ah, whn[...], preferred_element_type=jnp.float32)
         + ball[...])
    cv = cprev[...]
    i_g = jax.nn.sigmoid(g[:, 0:D] + wcj[0:1, :] * cv)
    f_g = jax.nn.sigmoid(g[:, D:2 * D] + wcj[1:2, :] * cv)
    c_tld = jnp.tanh(g[:, 2 * D:3 * D])
    ncv = f_g * cv + i_g * c_tld
    o_g = jax.nn.sigmoid(g[:, 3 * D:4 * D] + wcj[2:3, :] * ncv)
    nhv = o_g * jnp.tanh(ncv)
    nh[...] = nhv
    ncell[...] = ncv
    if final:
        out[...] = (jnp.dot(nhv, wout[...], preferred_element_type=jnp.float32)
                    + bout[...])


def _make_gate(final):
    rowspec = pl.BlockSpec((BLK, D), lambda i: (i, 0))
    pspec = pl.BlockSpec((NC, BLK, D), lambda i: (0, i, 0))
    full = lambda shape: pl.BlockSpec(shape, lambda i: tuple(0 for _ in shape))
    in_specs = [
        rowspec, rowspec, rowspec,
        pspec, pspec,
        pspec,
        full((D, 4 * D)), full((D, 4 * D)),
        full((D, 4 * D)), full((D, 4 * D)),
        full((3, D)), full((1, 4 * D)),
    ]
    n_out = 2
    out_shapes = [jax.ShapeDtypeStruct((N, D), jnp.float32)] * 2
    if final:
        in_specs += [full((D, D)), full((1, D))]
        n_out = 3
        out_shapes = out_shapes + [jax.ShapeDtypeStruct((N, D), jnp.float32)]
    return pl.pallas_call(
        functools.partial(_gate_body, final),
        grid=(GRID,),
        in_specs=in_specs,
        out_specs=[rowspec] * n_out,
        out_shape=out_shapes,
    )


_gate_c = functools.lru_cache(None)(_make_gate)


def kernel(x, hidden_states, cell_states, edge_index, Wx_self, Wx_neigh, bx,
           Wh_self, Wh_neigh, bh, wc, bias_gates, Wout, bout):
    src = edge_index[0]
    dst = edge_index[1]
    x0, x1 = x[0], x[1]
    h0, h1 = hidden_states[0], hidden_states[1]
    c0, c1 = cell_states[0], cell_states[1]

    def cat(Wq, j):
        return jnp.transpose(Wq[j], (1, 0, 2)).reshape(D, 4 * D)
    wxs = [cat(Wx_self, j) for j in range(L)]
    wxn = [cat(Wx_neigh, j) for j in range(L)]
    whs = [cat(Wh_self, j) for j in range(L)]
    whn = [cat(Wh_neigh, j) for j in range(L)]
    ball = [(bx[j] + bh[j] + bias_gates[j]).reshape(1, 4 * D)
            for j in range(L)]
    bout2 = bout.reshape(1, D)

    gate = _gate_c(False)
    gate_final = _gate_c(True)

    p_x0, p_x1, p_h0, p_h1, cntp = _agg(4, True)(x0, x1, h0, h1, src, dst)

    nh00, nc00 = gate(x0, h0, c0, p_x0, p_h0, cntp,
                      wxs[0], wxn[0], whs[0], whn[0], wc[0], ball[0])
    (p_nh00,) = _agg(1, False)(nh00, src, dst)
    nh10, nc10, out0 = gate_final(nh00, h1, c1, p_nh00, p_h1, cntp,
                                  wxs[1], wxn[1], whs[1], whn[1], wc[1],
                                  ball[1], Wout, bout2)
    nh01, nc01 = gate(x1, nh00, nc00, p_x1, p_nh00, cntp,
                      wxs[0], wxn[0], whs[0], whn[0], wc[0], ball[0])
    p_nh10, p_nh01 = _agg(2, False)(nh10, nh01, src, dst)
    nh11, nc11, out1 = gate_final(nh01, nh10, nc10, p_nh01, p_nh10, cntp,
                                  wxs[1], wxn[1], whs[1], whn[1], wc[1],
                                  ball[1], Wout, bout2)

    return (jnp.stack([out0, out1], axis=0),
            jnp.stack([nh01, nh11], axis=0),
            jnp.stack([nc01, nc11], axis=0))


# speedup vs baseline: 1.2566x; 1.2566x over previous
import functools

import jax
import jax.numpy as jnp
from jax import lax
from jax.experimental import pallas as pl
from jax.experimental.pallas import tpu as pltpu
from jax.experimental.pallas import tpu_sc as plsc

N = 10000
E = 320000
D = 128
L = 2
T = 2

NC = 2
NS = 16
NW = NC * NS
NP = 10240
RPT = NP // NS
EPW = E // NW
CH = 80
NCH = EPW // CH
RZ = 80
BLK = 1000
GRID = N // BLK


def _make_agg(num_feats, with_count):
    mesh = plsc.VectorSubcoreMesh(core_axis_name="c", subcore_axis_name="s")
    out_type = [jax.ShapeDtypeStruct((NC, NP, D), jnp.float32)
                for _ in range(num_feats + (1 if with_count else 0))]
    scratch = [
        pltpu.VMEM((EPW,), jnp.int32),
        pltpu.VMEM((EPW,), jnp.int32),
        pltpu.VMEM((CH, D), jnp.float32),
        pltpu.VMEM((CH, D), jnp.float32),
        pltpu.VMEM_SHARED((NP, D), jnp.float32),
        pltpu.SemaphoreType.DMA,
        pltpu.SemaphoreType.DMA,
    ]

    @functools.partial(pl.kernel, mesh=mesh, out_type=tuple(out_type),
                       scratch_types=tuple(scratch))
    def agg(*refs):
        feats = refs[:num_feats]
        src_h = refs[num_feats]
        dst_h = refs[num_feats + 1]
        n_out = num_feats + (1 if with_count else 0)
        outs = refs[num_feats + 2: num_feats + 2 + n_out]
        srcb, dstb, rows0, rows1, accum, sg0, sg1 = \
            refs[num_feats + 2 + n_out:]

        c = lax.axis_index("c")
        s = lax.axis_index("s")
        wid = s * NC + c
        ebase = wid * EPW
        base = s * RPT

        pltpu.sync_copy(src_h.at[pl.ds(ebase, EPW)], srcb)
        pltpu.sync_copy(dst_h.at[pl.ds(ebase, EPW)], dstb)

        def fill_rows(rb, val):
            def zrow(i, _):
                for l in range(D // 16):
                    rb[i, pl.ds(l * 16, 16)] = jnp.full((16,), val,
                                                        jnp.float32)
                return 0
            lax.fori_loop(0, RZ, zrow, 0)

        def zero_stripe():
            fill_rows(rows0, 0.0)
            for k in range(RPT // RZ):
                pltpu.sync_copy(rows0, accum.at[pl.ds(base + k * RZ, RZ)])

        def gather_start(f, j, rb, sg):
            pltpu.async_copy(feats[f].at[srcb.at[pl.ds(j * CH, CH)]], rb, sg)

        def gather_wait(f, j, rb, sg):
            pltpu.make_async_copy(
                feats[f].at[srcb.at[pl.ds(j * CH, CH)]], rb, sg).wait()

        zero_stripe()
        plsc.subcore_barrier()

        for f in range(n_out):
            ones = f >= num_feats
            if ones:
                fill_rows(rows0, 1.0)
                fill_rows(rows1, 1.0)

            def step(f, j, rb, sg):
                if not ones:
                    gather_wait(f, j, rb, sg)
                pltpu.sync_copy(rb, accum.at[dstb.at[pl.ds(j * CH, CH)]],
                                add=True)

            def start(f, j, rb, sg):
                if not ones:
                    gather_start(f, j, rb, sg)

            start(f, 0, rows0, sg0)

            def quad(i, _):
                j0 = 4 * i
                for q in range(4):
                    rb, sg = (rows0, sg0) if q % 2 == 0 else (rows1, sg1)
                    nrb, nsg = (rows1, sg1) if q % 2 == 0 else (rows0, sg0)
                    start(f, j0 + q + 1, nrb, nsg)
                    step(f, j0 + q, rb, sg)
                return 0
            lax.fori_loop(0, (NCH - 1) // 4, quad, 0)
            step(f, NCH - 1, rows0, sg0)
            plsc.subcore_barrier()

            for k in range(RPT // RZ):
                r0 = base + k * RZ
                pltpu.sync_copy(accum.at[pl.ds(r0, RZ)], rows0)
                pltpu.sync_copy(rows0, outs[f].at[c, pl.ds(r0, RZ)])
            if f + 1 < n_out:
                zero_stripe()
                plsc.subcore_barrier()

    return agg


_agg = functools.lru_cache(None)(_make_agg)


def _gate_body(final, *refs):
    if final:
        (inp, h, cprev, axp, ahp, cntp, wxs, wxn, whs, whn, wcj, ball,
         wout, bout, nh, ncell, out) = refs
    else:
        (inp, h, cprev, axp, ahp, cntp, wxs, wxn, whs, whn, wcj, ball,
         nh, ncell) = refs
    cnt = cntp[0, :, 0:1] + cntp[1, :, 0:1]
    inv = 1.0 / jnp.maximum(cnt, 1.0)
    ax = (axp[0] + axp[1]) * inv
    ah = (ahp[0] + ahp[1]) * inv
    g = (jnp.dot(inp[...], wxs[...], preferred_element_type=jnp.float32)
         + jnp.dot(ax, wxn[...], preferred_element_type=jnp.float32)
         + jnp.dot(h[...], whs[...], preferred_element_type=jnp.float32)
         + jnp.dot(ah, whn[...], preferred_element_type=jnp.float32)
         + ball[...])
    cv = cprev[...]
    i_g = jax.nn.sigmoid(g[:, 0:D] + wcj[0:1, :] * cv)
    f_g = jax.nn.sigmoid(g[:, D:2 * D] + wcj[1:2, :] * cv)
    c_tld = jnp.tanh(g[:, 2 * D:3 * D])
    ncv = f_g * cv + i_g * c_tld
    o_g = jax.nn.sigmoid(g[:, 3 * D:4 * D] + wcj[2:3, :] * ncv)
    nhv = o_g * jnp.tanh(ncv)
    nh[...] = nhv
    ncell[...] = ncv
    if final:
        out[...] = (jnp.dot(nhv, wout[...], preferred_element_type=jnp.float32)
                    + bout[...])


def _make_gate(final):
    rowspec = pl.BlockSpec((BLK, D), lambda i: (i, 0))
    pspec = pl.BlockSpec((NC, BLK, D), lambda i: (0, i, 0))
    full = lambda shape: pl.BlockSpec(shape, lambda i: tuple(0 for _ in shape))
    in_specs = [
        rowspec, rowspec, rowspec,
        pspec, pspec,
        pspec,
        full((D, 4 * D)), full((D, 4 * D)),
        full((D, 4 * D)), full((D, 4 * D)),
        full((3, D)), full((1, 4 * D)),
    ]
    n_out = 2
    out_shapes = [jax.ShapeDtypeStruct((N, D), jnp.float32)] * 2
    if final:
        in_specs += [full((D, D)), full((1, D))]
        n_out = 3
        out_shapes = out_shapes + [jax.ShapeDtypeStruct((N, D), jnp.float32)]
    return pl.pallas_call(
        functools.partial(_gate_body, final),
        grid=(GRID,),
        in_specs=in_specs,
        out_specs=[rowspec] * n_out,
        out_shape=out_shapes,
    )


_gate_c = functools.lru_cache(None)(_make_gate)


def kernel(x, hidden_states, cell_states, edge_index, Wx_self, Wx_neigh, bx,
           Wh_self, Wh_neigh, bh, wc, bias_gates, Wout, bout):
    src = edge_index[0]
    dst = edge_index[1]
    x0, x1 = x[0], x[1]
    h0, h1 = hidden_states[0], hidden_states[1]
    c0, c1 = cell_states[0], cell_states[1]

    def cat(Wq, j):
        return jnp.transpose(Wq[j], (1, 0, 2)).reshape(D, 4 * D)
    wxs = [cat(Wx_self, j) for j in range(L)]
    wxn = [cat(Wx_neigh, j) for j in range(L)]
    whs = [cat(Wh_self, j) for j in range(L)]
    whn = [cat(Wh_neigh, j) for j in range(L)]
    ball = [(bx[j] + bh[j] + bias_gates[j]).reshape(1, 4 * D)
            for j in range(L)]
    bout2 = bout.reshape(1, D)

    gate = _gate_c(False)
    gate_final = _gate_c(True)

    p_x0, p_x1, p_h0, p_h1, cntp = _agg(4, True)(x0, x1, h0, h1, src, dst)

    nh00, nc00 = gate(x0, h0, c0, p_x0, p_h0, cntp,
                      wxs[0], wxn[0], whs[0], whn[0], wc[0], ball[0])
    (p_nh00,) = _agg(1, False)(nh00, src, dst)
    nh10, nc10, out0 = gate_final(nh00, h1, c1, p_nh00, p_h1, cntp,
                                  wxs[1], wxn[1], whs[1], whn[1], wc[1],
                                  ball[1], Wout, bout2)
    nh01, nc01 = gate(x1, nh00, nc00, p_x1, p_nh00, cntp,
                      wxs[0], wxn[0], whs[0], whn[0], wc[0], ball[0])
    p_nh10, p_nh01 = _agg(2, False)(nh10, nh01, src, dst)
    nh11, nc11, out1 = gate_final(nh01, nh10, nc10, p_nh01, p_nh10, cntp,
                                  wxs[1], wxn[1], whs[1], whn[1], wc[1],
                                  ball[1], Wout, bout2)

    return (jnp.stack([out0, out1], axis=0),
            jnp.stack([nh01, nh11], axis=0),
            jnp.stack([nc01, nc11], axis=0))
